# Initial kernel scaffold; baseline (speedup 1.0000x reference)
import jax
import jax.numpy as jnp
from jax.experimental import pallas as pl


def kernel(x, edge_index, edge_attr, y, W0, b0, W1, b1, W2, b2, W3, b3):
    raise NotImplementedError("write your pallas kernel here")


# speedup vs baseline: 7.3213x; 7.3213x over previous
import functools

import jax
import jax.numpy as jnp
from jax import lax
from jax.experimental import pallas as pl
from jax.experimental.pallas import tpu as pltpu, tpu_sc as plsc

_N = 50000
_E = 800000
_CH = 128
_K2 = 8
_SUPE = _K2 * _CH
_TILES = 16
_NSUP = 50
_EPT = _NSUP * _SUPE
_EP = _EPT * _TILES
_NP = 51200
_RPT = _NP // _TILES
_BN = 2000
_BNP = 400


def _make_spmm(Dc):
    mesh = plsc.VectorSubcoreMesh(core_axis_name="c", subcore_axis_name="s")

    @functools.partial(
        pl.kernel,
        out_type=jax.ShapeDtypeStruct((2 * _NP, Dc), jnp.float32),
        mesh=mesh,
        compiler_params=pltpu.CompilerParams(use_tc_tiling_on_sc=False),
        scratch_types=[
            pltpu.VMEM((_K2, _CH), jnp.int32),
            pltpu.VMEM((_K2, _CH), jnp.int32),
            pltpu.VMEM((_K2, _CH), jnp.float32),
            pltpu.VMEM((_SUPE, Dc), jnp.float32),
            pltpu.VMEM_SHARED((_NP, Dc), jnp.float32),
            pltpu.SemaphoreType.DMA,
        ],
    )
    def spmm(rowp_hbm, col_hbm, ew_hbm, hwb_hbm, zeros_hbm, out_hbm,
             rbuf, cbuf, wbuf, gbuf, acc, sem):
        cid = lax.axis_index("c")
        sid = lax.axis_index("s")
        pltpu.sync_copy(zeros_hbm, acc.at[pl.ds(sid * _RPT, _RPT)])
        plsc.subcore_barrier()

        rsup0 = (cid * _TILES + sid) * _NSUP
        csup0 = sid * _NSUP

        def body(s, carry):
            pltpu.sync_copy(rowp_hbm.at[rsup0 + s], rbuf)
            pltpu.sync_copy(col_hbm.at[csup0 + s], cbuf)
            pltpu.sync_copy(ew_hbm.at[csup0 + s], wbuf)
            descs = [
                pltpu.async_copy(hwb_hbm.at[rbuf.at[j]],
                                 gbuf.at[pl.ds(j * _CH, _CH)], sem)
                for j in range(_K2)
            ]
            for dsc in descs:
                dsc.wait()

            def scale(jj, c2):
                for g in range(_CH // 16):
                    wv = wbuf[jj, pl.ds(g * 16, 16)]
                    for j in range(16):
                        w = wv[j]
                        e = jj * _CH + g * 16 + j
                        for b16 in range(Dc // 16):
                            sl = pl.ds(b16 * 16, 16)
                            gbuf[e, sl] = gbuf[e, sl] * w
                return c2

            lax.fori_loop(0, _K2, scale, 0)
            for j in range(_K2):
                pltpu.async_copy(gbuf.at[pl.ds(j * _CH, _CH)],
                                 acc.at[cbuf.at[j]], sem, add=True).wait()
            return carry

        lax.fori_loop(0, _NSUP, body, 0)
        plsc.subcore_barrier()
        r0 = sid * _RPT
        pltpu.sync_copy(acc.at[pl.ds(r0, _RPT)],
                        out_hbm.at[pl.ds(cid * _NP + r0, _RPT)])

    return spmm


_SPMM = {Dc: _make_spmm(Dc) for Dc in (16,)}


def _tc_pre(h, Wb, dis, Dc):
    n, k = h.shape

    def body(h_ref, w_ref, d_ref, o_ref):
        w = w_ref[...][0]
        hw = jnp.dot(h_ref[...], w, preferred_element_type=jnp.float32,
                     precision=lax.Precision.HIGHEST)
        o_ref[...] = hw * d_ref[...]

    return pl.pallas_call(
        body,
        grid=(2, n // _BN),
        in_specs=[
            pl.BlockSpec((_BN, k), lambda c, j: (j, 0)),
            pl.BlockSpec((1, k, Dc), lambda c, j: (c, 0, 0)),
            pl.BlockSpec((_BN, 1), lambda c, j: (j, 0)),
        ],
        out_specs=pl.BlockSpec((_BN, Dc), lambda c, j: (c * (n // _BN) + j, 0)),
        out_shape=jax.ShapeDtypeStruct((2 * n, Dc), jnp.float32),
    )(h, Wb, dis)


def _tc_post(groups, dis, bp, D, relu):
    dcs = [g[2] for g in groups]

    def body(*refs):
        o_ref = refs[-1]
        d_ref, b_ref = refs[-3], refs[-2]
        d = d_ref[...]
        bb = b_ref[...]
        off = 0
        for gi in range(len(dcs)):
            dc = dcs[gi]
            a_lo, a_hi, h_lo, h_hi = refs[4 * gi: 4 * gi + 4]
            for a, h in ((a_lo, h_lo), (a_hi, h_hi)):
                w = min(off + dc, D) - off
                if w > 0:
                    y = (a[...] + h[...]) * d + bb[:, off:off + dc]
                    if relu:
                        y = jax.nn.leaky_relu(y)
                    o_ref[:, off:off + w] = y[:, :w]
                off += dc

    nbp = _NP // _BNP
    in_specs = []
    args = []
    nbh = _N // _BNP
    for accb, hwb, dc in groups:
        spec_lo = pl.BlockSpec((_BNP, dc), lambda j: (j, 0))
        spec_hi_a = pl.BlockSpec((_BNP, dc), lambda j, _n=nbp: (_n + j, 0))
        spec_hi_h = pl.BlockSpec((_BNP, dc), lambda j, _n=nbh: (_n + j, 0))
        in_specs += [spec_lo, spec_hi_a, spec_lo, spec_hi_h]
        args += [accb, accb, hwb, hwb]
    in_specs += [
        pl.BlockSpec((_BNP, 1), lambda j: (j, 0)),
        pl.BlockSpec((1, bp.shape[1]), lambda j: (0, 0)),
    ]
    args += [dis, bp]
    return pl.pallas_call(
        body,
        grid=(_N // _BNP,),
        in_specs=in_specs,
        out_specs=pl.BlockSpec((_BNP, D), lambda j: (j, 0)),
        out_shape=jax.ShapeDtypeStruct((_N, D), jnp.float32),
    )(*args)


def _tc_dis(degcol):
    def body(a_ref, o_ref):
        d = a_ref[...] + 1.0
        o_ref[...] = jnp.where(d > 0, lax.rsqrt(jnp.maximum(d, 1e-12)), 0.0)

    return pl.pallas_call(
        body,
        grid=(_N // _BN,),
        in_specs=[pl.BlockSpec((_BN, 1), lambda j: (j, 0))],
        out_specs=pl.BlockSpec((_BN, 1), lambda j: (j, 0)),
        out_shape=jax.ShapeDtypeStruct((_N, 1), jnp.float32),
    )(degcol)


def _block_w(W, b, col_groups):
    k, d = W.shape
    tot = 2 * sum(dc for _, dc in col_groups)
    Wp = jnp.zeros((k, tot), W.dtype).at[:, :d].set(W)
    bpad = jnp.zeros((1, tot), b.dtype).at[0, :d].set(b)
    blocks = []
    for c0, dc in col_groups:
        blocks.append(jnp.stack([Wp[:, c0:c0 + dc], Wp[:, c0 + dc:c0 + 2 * dc]]))
    return blocks, bpad


def _conv(h, Wblocks, bp, dis, rowp, colp, ewp, zeros16, D, relu):
    groups = []
    for Wb in Wblocks:
        hwb = _tc_pre(h, Wb, dis, 16)
        accb = _SPMM[16](rowp, colp, ewp, hwb, zeros16)
        groups.append((accb, hwb, 16))
    return _tc_post(groups, dis, bp, D, relu)


def kernel(x, edge_index, edge_attr, y, W0, b0, W1, b1, W2, b2, W3, b3):
    row = edge_index[0]
    col = edge_index[1]
    pad = _EP - _E
    rowz = jnp.concatenate([row, jnp.zeros((pad,), jnp.int32)])
    colz = jnp.concatenate([col, jnp.zeros((pad,), jnp.int32)])
    ewz = jnp.concatenate([edge_attr, jnp.zeros((pad,), jnp.float32)])
    rowp = jnp.concatenate([rowz, rowz + _N]).reshape(2 * _TILES * _NSUP, _K2, _CH)
    colp = colz.reshape(_TILES * _NSUP, _K2, _CH)
    ewp = ewz.reshape(_TILES * _NSUP, _K2, _CH)
    zeros16 = jnp.zeros((_RPT, 16), jnp.float32)

    W0b, b0p = _block_w(W0, b0, [(0, 16)])
    W1b, b1p = _block_w(W1, b1, [(0, 16)])
    W2b, b2p = _block_w(W2, b2, [(0, 16), (32, 16), (64, 16)])
    W3b, b3p = _block_w(W3, b3, [(0, 16)])

    ones_tab = jnp.ones((2 * _N, 16), jnp.float32)
    degb = _SPMM[16](rowp, colp, ewp, ones_tab, zeros16)
    dis = _tc_dis(degb[:_N, 0:1])

    t_future = y.shape[1]
    x_init = x
    preds = []
    for _ in range(t_future):
        h1 = _conv(x_init, W0b, b0p, dis, rowp, colp, ewp, zeros16, 32, True)
        h2 = _conv(h1, W1b, b1p, dis, rowp, colp, ewp, zeros16, 32, True)
        temp = jnp.concatenate([x_init, h1, h2], axis=1)
        x_temp = _conv(temp, W2b, b2p, dis, rowp, colp, ewp, zeros16, 76, True)
        y_pred = _conv(x_temp, W3b, b3p, dis, rowp, colp, ewp, zeros16, 1, False)
        preds.append(y_pred)
        x_init = jnp.concatenate([x_init[:, 1:], y_pred], axis=1)
    return jnp.concatenate(preds, axis=1)
